# baseline (speedup 1.0000x reference)
import functools
import math

import jax
import jax.numpy as jnp
from jax import lax
from jax.experimental import pallas as pl
from jax.experimental.pallas import tpu as pltpu
from jax.experimental.pallas import tpu_sc as plsc

N_F0_BINS = 256
V1_DIM = 768
HIDDEN_DIM = 512
F0_MIN = 50.0
F0_MAX = 1100.0

_MEL_MIN = 1127.0 * math.log(1.0 + F0_MIN / 700.0)
_MEL_MAX = 1127.0 * math.log(1.0 + F0_MAX / 700.0)

_NC = 2
_NS = 16
_NW = _NC * _NS


def _prep_body(f0_ref, emb_ref, w1_ref, b1_ref, w2_ref, b2_ref,
               bins_ref, table_ref):
    f0 = f0_ref[...]
    f0_mel = 1127.0 * jnp.log(1.0 + f0 / 700.0)
    f0_mel = jnp.where(
        f0_mel > 0.0,
        (f0_mel - _MEL_MIN) * (N_F0_BINS - 2) / (_MEL_MAX - _MEL_MIN) + 1.0,
        f0_mel,
    )
    f0_mel = jnp.where(f0_mel <= 1.0, 1.0, f0_mel)
    f0_mel = jnp.where(f0_mel > N_F0_BINS - 1, float(N_F0_BINS - 1), f0_mel)
    bins_ref[...] = (f0_mel + 0.5).astype(jnp.int32)

    h = jnp.dot(emb_ref[...], w1_ref[...], preferred_element_type=jnp.float32)
    h = h + b1_ref[...]
    h = 0.5 * h * (1.0 + lax.erf(h * (1.0 / math.sqrt(2.0))))
    t = jnp.dot(h, w2_ref[...], preferred_element_type=jnp.float32)
    table_ref[...] = t + b2_ref[...]


_NBUF = 2
_LANES = 16


def _make_sc_gather(n_rows, d, chunk):
    dh = d // 2
    n_per_p = n_rows // _NS
    n_chunks = n_per_p // chunk
    groups = chunk // _LANES
    mesh = plsc.VectorSubcoreMesh(core_axis_name="c", subcore_axis_name="s")

    @functools.partial(
        pl.kernel,
        mesh=mesh,
        out_type=jax.ShapeDtypeStruct((n_rows, d), jnp.float32),
        scratch_types=[
            pltpu.VMEM((n_per_p,), jnp.int32),
            pltpu.VMEM((N_F0_BINS, dh + 1), jnp.float32),
            pltpu.VMEM((_NBUF, chunk, dh + 1), jnp.float32),
        ]
        + [pltpu.SemaphoreType.DMA] * _NBUF,
        compiler_params=pltpu.CompilerParams(needs_layout_passes=False),
    )
    def gather_kernel(table_hbm, bins_hbm, out_hbm, idx_v, table_v,
                      stage_v, *ssems):
        sid = lax.axis_index("s")
        half = lax.axis_index("c")
        fbase = half * dh
        rbase = sid * n_per_p
        pltpu.sync_copy(table_hbm.at[:, pl.ds(fbase, dh)],
                        table_v.at[:, pl.ds(0, dh)])
        pltpu.sync_copy(bins_hbm.at[pl.ds(rbase, n_per_p)], idx_v)

        row_ids = [
            jax.lax.iota(jnp.int32, _LANES) + gg * _LANES
            for gg in range(groups)
        ]

        def store_chunk(ci, b):
            return pltpu.make_async_copy(
                stage_v.at[b, :, pl.ds(0, dh)],
                out_hbm.at[pl.ds(rbase + ci * chunk, chunk),
                           pl.ds(fbase, dh)],
                ssems[b])

        def fill_chunk(ci, b):
            for gg in range(groups):
                bins16 = idx_v[pl.ds(ci * chunk + gg * _LANES, _LANES)]

                @plsc.parallel_loop(0, dh, unroll=8)
                def _(f):
                    fvec = jnp.full((_LANES,), f, jnp.int32)
                    vals = plsc.load_gather(table_v, [bins16, fvec])
                    plsc.store_scatter(stage_v.at[b], [row_ids[gg], fvec],
                                       vals)

        def body(j, _):
            for b in range(_NBUF):
                ci = j * _NBUF + b

                @pl.when(j >= 1)
                def _():
                    store_chunk(ci - _NBUF, b).wait()

                fill_chunk(ci, b)
                store_chunk(ci, b).start()
            return 0

        lax.fori_loop(0, n_chunks // _NBUF, body, 0)
        for b in range(_NBUF):
            store_chunk(n_chunks - _NBUF + b, b).wait()

    return gather_kernel


_TC_TILE = 512


def _onehot_body(bins_ref, table_ref, out_ref):
    b = bins_ref[0]
    onehot = (b[:, :, None] == lax.broadcasted_iota(
        jnp.int32, (1, _TC_TILE, N_F0_BINS), 2)).astype(jnp.bfloat16)
    out_ref[0] = jnp.dot(onehot[0], table_ref[...],
                         preferred_element_type=jnp.float32)


def kernel(f0, emb, W1, b1, W2, b2):
    B, T = f0.shape
    d = W2.shape[1]

    bins, table = pl.pallas_call(
        _prep_body,
        out_shape=(
            jax.ShapeDtypeStruct((B, T), jnp.int32),
            jax.ShapeDtypeStruct((N_F0_BINS, d), jnp.float32),
        ),
    )(f0, emb, W1, b1.reshape(1, -1), W2, b2.reshape(1, -1))

    n_rows = B * T
    out = _make_sc_gather(n_rows, d, chunk=32)(table, bins.reshape(n_rows))
    return out.reshape(B, T, d)


# speedup vs baseline: 1.1209x; 1.1209x over previous
import functools
import math

import jax
import jax.numpy as jnp
from jax import lax
from jax.experimental import pallas as pl
from jax.experimental.pallas import tpu as pltpu
from jax.experimental.pallas import tpu_sc as plsc

N_F0_BINS = 256
V1_DIM = 768
HIDDEN_DIM = 512
F0_MIN = 50.0
F0_MAX = 1100.0

_MEL_MIN = 1127.0 * math.log(1.0 + F0_MIN / 700.0)
_MEL_MAX = 1127.0 * math.log(1.0 + F0_MAX / 700.0)

_NC = 2
_NS = 16
_NW = _NC * _NS


def _prep_body(f0_ref, emb_ref, w1_ref, b1_ref, w2_ref, b2_ref,
               bins_ref, table_ref):
    f0 = f0_ref[...]
    f0_mel = 1127.0 * jnp.log(1.0 + f0 / 700.0)
    f0_mel = jnp.where(
        f0_mel > 0.0,
        (f0_mel - _MEL_MIN) * (N_F0_BINS - 2) / (_MEL_MAX - _MEL_MIN) + 1.0,
        f0_mel,
    )
    f0_mel = jnp.where(f0_mel <= 1.0, 1.0, f0_mel)
    f0_mel = jnp.where(f0_mel > N_F0_BINS - 1, float(N_F0_BINS - 1), f0_mel)
    bins_ref[...] = (f0_mel + 0.5).astype(jnp.int32)

    h = jnp.dot(emb_ref[...], w1_ref[...], preferred_element_type=jnp.float32)
    h = h + b1_ref[...]
    h = 0.5 * h * (1.0 + lax.erf(h * (1.0 / math.sqrt(2.0))))
    t = jnp.dot(h, w2_ref[...], preferred_element_type=jnp.float32)
    table_ref[...] = t + b2_ref[...]


_NBUF = 2
_LANES = 16


def _make_sc_gather(n_rows, d, chunk):
    dh = d // 2
    n_per_p = n_rows // _NS
    n_chunks = n_per_p // chunk
    groups = chunk // _LANES
    mesh = plsc.VectorSubcoreMesh(core_axis_name="c", subcore_axis_name="s")

    @functools.partial(
        pl.kernel,
        mesh=mesh,
        out_type=jax.ShapeDtypeStruct((n_rows, d), jnp.float32),
        scratch_types=[
            pltpu.VMEM((n_per_p,), jnp.int32),
            pltpu.VMEM((N_F0_BINS, dh + 1), jnp.float32),
            pltpu.VMEM((_NBUF, chunk, dh + 1), jnp.float32),
        ]
        + [pltpu.SemaphoreType.DMA] * _NBUF,
        compiler_params=pltpu.CompilerParams(needs_layout_passes=False),
    )
    def gather_kernel(table_hbm, bins_hbm, out_hbm, idx_v, table_v,
                      stage_v, *ssems):
        sid = lax.axis_index("s")
        half = lax.axis_index("c")
        fbase = half * dh
        rbase = sid * n_per_p
        pltpu.sync_copy(table_hbm.at[:, pl.ds(fbase, dh)],
                        table_v.at[:, pl.ds(0, dh)])
        pltpu.sync_copy(bins_hbm.at[pl.ds(rbase, n_per_p)], idx_v)

        row_ids = [
            jax.lax.iota(jnp.int32, _LANES) + gg * _LANES
            for gg in range(groups)
        ]

        def store_chunk(ci, b):
            return pltpu.make_async_copy(
                stage_v.at[b, :, pl.ds(0, dh)],
                out_hbm.at[pl.ds(rbase + ci * chunk, chunk),
                           pl.ds(fbase, dh)],
                ssems[b])

        def fill_chunk(ci, b):
            for gg in range(groups):
                bins16 = idx_v[pl.ds(ci * chunk + gg * _LANES, _LANES)]

                @plsc.parallel_loop(0, dh, step=_LANES, unroll=1)
                def _(f0):
                    for t in range(_LANES):
                        fvec = jnp.full((_LANES,), f0 + t, jnp.int32)
                        vals = plsc.load_gather(table_v, [bins16, fvec])
                        plsc.store_scatter(stage_v.at[b],
                                           [row_ids[gg], fvec], vals)

        def body(j, _):
            for b in range(_NBUF):
                ci = j * _NBUF + b

                @pl.when(j >= 1)
                def _():
                    store_chunk(ci - _NBUF, b).wait()

                fill_chunk(ci, b)
                store_chunk(ci, b).start()
            return 0

        lax.fori_loop(0, n_chunks // _NBUF, body, 0)
        for b in range(_NBUF):
            store_chunk(n_chunks - _NBUF + b, b).wait()

    return gather_kernel


_TC_TILE = 512


def _onehot_body(bins_ref, table_ref, out_ref):
    b = bins_ref[0]
    onehot = (b[:, :, None] == lax.broadcasted_iota(
        jnp.int32, (1, _TC_TILE, N_F0_BINS), 2)).astype(jnp.bfloat16)
    out_ref[0] = jnp.dot(onehot[0], table_ref[...],
                         preferred_element_type=jnp.float32)


def kernel(f0, emb, W1, b1, W2, b2):
    B, T = f0.shape
    d = W2.shape[1]

    bins, table = pl.pallas_call(
        _prep_body,
        out_shape=(
            jax.ShapeDtypeStruct((B, T), jnp.int32),
            jax.ShapeDtypeStruct((N_F0_BINS, d), jnp.float32),
        ),
    )(f0, emb, W1, b1.reshape(1, -1), W2, b2.reshape(1, -1))

    n_rows = B * T
    out = _make_sc_gather(n_rows, d, chunk=32)(table, bins.reshape(n_rows))
    return out.reshape(B, T, d)


# speedup vs baseline: 5.7990x; 5.1736x over previous
import functools
import math

import jax
import jax.numpy as jnp
from jax import lax
from jax.experimental import pallas as pl
from jax.experimental.pallas import tpu as pltpu
from jax.experimental.pallas import tpu_sc as plsc

N_F0_BINS = 256
V1_DIM = 768
HIDDEN_DIM = 512
F0_MIN = 50.0
F0_MAX = 1100.0

_MEL_MIN = 1127.0 * math.log(1.0 + F0_MIN / 700.0)
_MEL_MAX = 1127.0 * math.log(1.0 + F0_MAX / 700.0)

_NC = 2
_NS = 16
_NW = _NC * _NS


def _prep_body(f0_ref, emb_ref, w1_ref, b1_ref, w2_ref, b2_ref,
               bins_ref, table_ref):
    f0 = f0_ref[...]
    f0_mel = 1127.0 * jnp.log(1.0 + f0 / 700.0)
    f0_mel = jnp.where(
        f0_mel > 0.0,
        (f0_mel - _MEL_MIN) * (N_F0_BINS - 2) / (_MEL_MAX - _MEL_MIN) + 1.0,
        f0_mel,
    )
    f0_mel = jnp.where(f0_mel <= 1.0, 1.0, f0_mel)
    f0_mel = jnp.where(f0_mel > N_F0_BINS - 1, float(N_F0_BINS - 1), f0_mel)
    bins_ref[...] = (f0_mel + 0.5).astype(jnp.int32)

    h = jnp.dot(emb_ref[...], w1_ref[...], preferred_element_type=jnp.float32)
    h = h + b1_ref[...]
    h = 0.5 * h * (1.0 + lax.erf(h * (1.0 / math.sqrt(2.0))))
    t = jnp.dot(h, w2_ref[...], preferred_element_type=jnp.float32)
    table_ref[...] = t + b2_ref[...]


_NBUF = 4


def _make_sc_gather(n_rows, d, chunk):
    n_per_w = n_rows // _NW
    n_chunks = n_per_w // chunk
    assert n_chunks % _NBUF == 0
    mesh = plsc.VectorSubcoreMesh(core_axis_name="c", subcore_axis_name="s")

    @functools.partial(
        pl.kernel,
        mesh=mesh,
        out_type=jax.ShapeDtypeStruct((n_rows, d), jnp.float32),
        scratch_types=[
            pltpu.VMEM((n_per_w,), jnp.int32),
            pltpu.VMEM((_NBUF, chunk, d), jnp.float32),
        ]
        + [pltpu.SemaphoreType.DMA] * (2 * _NBUF),
    )
    def gather_kernel(table_hbm, bins_hbm, out_hbm, idx_v, rows_v, *sems):
        gsems, ssems = sems[:_NBUF], sems[_NBUF:]
        wid = lax.axis_index("s") * _NC + lax.axis_index("c")
        base = wid * n_per_w
        pltpu.sync_copy(bins_hbm.at[pl.ds(base, n_per_w)], idx_v)

        def gather_chunk(off, b):
            return pltpu.make_async_copy(
                table_hbm.at[idx_v.at[pl.ds(off, chunk)]],
                rows_v.at[b], gsems[b])

        def store_chunk(off, b):
            return pltpu.make_async_copy(
                rows_v.at[b], out_hbm.at[pl.ds(base + off, chunk)], ssems[b])

        for b in range(_NBUF):
            gather_chunk(b * chunk, b).start()

        def body(j, _):
            for b in range(_NBUF):
                i = j * _NBUF + b
                off = i * chunk
                gather_chunk(off, b).wait()
                store_chunk(off, b).start()
                store_chunk(off, b).wait()

                @pl.when(j + 1 < n_chunks // _NBUF)
                def _():
                    gather_chunk(off + _NBUF * chunk, b).start()

            return 0

        lax.fori_loop(0, n_chunks // _NBUF, body, 0)

    return gather_kernel


_TC_TILE = 512


def _onehot_body(bins_ref, table_ref, out_ref):
    b = bins_ref[0]
    onehot = (b[:, :, None] == lax.broadcasted_iota(
        jnp.int32, (1, _TC_TILE, N_F0_BINS), 2)).astype(jnp.bfloat16)
    out_ref[0] = jnp.dot(onehot[0], table_ref[...],
                         preferred_element_type=jnp.float32)


def kernel(f0, emb, W1, b1, W2, b2):
    B, T = f0.shape
    d = W2.shape[1]

    bins, table = pl.pallas_call(
        _prep_body,
        out_shape=(
            jax.ShapeDtypeStruct((B, T), jnp.int32),
            jax.ShapeDtypeStruct((N_F0_BINS, d), jnp.float32),
        ),
    )(f0, emb, W1, b1.reshape(1, -1), W2, b2.reshape(1, -1))

    n_rows = B * T
    n_sc = 20480
    bins_flat = bins.reshape(n_rows)

    out_sc = _make_sc_gather(n_sc, d, chunk=32)(table, bins_flat[:n_sc])

    nt_all = n_rows // _TC_TILE
    nt_sc = n_sc // _TC_TILE
    nt = nt_all - nt_sc
    table_bf = table.astype(jnp.bfloat16)
    out_full = pl.pallas_call(
        _onehot_body,
        grid=(nt,),
        in_specs=[
            pl.BlockSpec((1, 1, _TC_TILE), lambda i: (i, 0, 0)),
            pl.BlockSpec((N_F0_BINS, d), lambda i: (0, 0)),
        ],
        out_specs=pl.BlockSpec((1, _TC_TILE, d), lambda i: (i + nt_sc, 0, 0)),
        out_shape=jax.ShapeDtypeStruct((nt_all, _TC_TILE, d), jnp.float32),
    )(bins_flat[n_sc:].reshape(nt, 1, _TC_TILE), table_bf)

    out = lax.dynamic_update_slice(out_full.reshape(n_rows, d), out_sc, (0, 0))
    return out.reshape(B, T, d)


# speedup vs baseline: 6.2543x; 1.0785x over previous
import functools
import math

import jax
import jax.numpy as jnp
from jax import lax
from jax.experimental import pallas as pl
from jax.experimental.pallas import tpu as pltpu
from jax.experimental.pallas import tpu_sc as plsc

N_F0_BINS = 256
V1_DIM = 768
HIDDEN_DIM = 512
F0_MIN = 50.0
F0_MAX = 1100.0

_MEL_MIN = 1127.0 * math.log(1.0 + F0_MIN / 700.0)
_MEL_MAX = 1127.0 * math.log(1.0 + F0_MAX / 700.0)

_NC = 2
_NS = 16
_NW = _NC * _NS


def _prep_body(f0_ref, emb_ref, w1_ref, b1_ref, w2_ref, b2_ref,
               bins_ref, table_ref):
    f0 = f0_ref[...]
    f0_mel = 1127.0 * jnp.log(1.0 + f0 / 700.0)
    f0_mel = jnp.where(
        f0_mel > 0.0,
        (f0_mel - _MEL_MIN) * (N_F0_BINS - 2) / (_MEL_MAX - _MEL_MIN) + 1.0,
        f0_mel,
    )
    f0_mel = jnp.where(f0_mel <= 1.0, 1.0, f0_mel)
    f0_mel = jnp.where(f0_mel > N_F0_BINS - 1, float(N_F0_BINS - 1), f0_mel)
    bins_ref[...] = (f0_mel + 0.5).astype(jnp.int32)

    h = jnp.dot(emb_ref[...], w1_ref[...], preferred_element_type=jnp.float32)
    h = h + b1_ref[...]
    h = 0.5 * h * (1.0 + lax.erf(h * (1.0 / math.sqrt(2.0))))
    t = jnp.dot(h, w2_ref[...], preferred_element_type=jnp.float32)
    table_ref[...] = t + b2_ref[...]


_NBUF = 4


def _make_sc_gather(n_rows, d, chunk):
    n_per_w = n_rows // _NW
    n_chunks = n_per_w // chunk
    assert n_chunks % _NBUF == 0
    mesh = plsc.VectorSubcoreMesh(core_axis_name="c", subcore_axis_name="s")

    @functools.partial(
        pl.kernel,
        mesh=mesh,
        out_type=jax.ShapeDtypeStruct((n_rows, d), jnp.float32),
        scratch_types=[
            pltpu.VMEM((n_per_w,), jnp.int32),
            pltpu.VMEM((_NBUF, chunk, d), jnp.float32),
        ]
        + [pltpu.SemaphoreType.DMA] * (2 * _NBUF),
    )
    def gather_kernel(table_hbm, bins_hbm, out_hbm, idx_v, rows_v, *sems):
        gsems, ssems = sems[:_NBUF], sems[_NBUF:]
        wid = lax.axis_index("s") * _NC + lax.axis_index("c")
        base = wid * n_per_w
        pltpu.sync_copy(bins_hbm.at[pl.ds(base, n_per_w)], idx_v)

        def gather_chunk(off, b):
            return pltpu.make_async_copy(
                table_hbm.at[idx_v.at[pl.ds(off, chunk)]],
                rows_v.at[b], gsems[b])

        def store_chunk(off, b):
            return pltpu.make_async_copy(
                rows_v.at[b], out_hbm.at[pl.ds(base + off, chunk)], ssems[b])

        for b in range(_NBUF):
            gather_chunk(b * chunk, b).start()

        def body(j, _):
            for b in range(_NBUF):
                i = j * _NBUF + b
                off = i * chunk
                gather_chunk(off, b).wait()
                store_chunk(off, b).start()
                store_chunk(off, b).wait()

                @pl.when(j + 1 < n_chunks // _NBUF)
                def _():
                    gather_chunk(off + _NBUF * chunk, b).start()

            return 0

        lax.fori_loop(0, n_chunks // _NBUF, body, 0)

    return gather_kernel


_TC_TILE = 2048


def _onehot_body(bins_ref, table_ref, out_ref):
    b = bins_ref[0]
    onehot = (b[:, :, None] == lax.broadcasted_iota(
        jnp.int32, (1, _TC_TILE, N_F0_BINS), 2)).astype(jnp.bfloat16)
    out_ref[0] = jnp.dot(onehot[0], table_ref[...],
                         preferred_element_type=jnp.float32)


def kernel(f0, emb, W1, b1, W2, b2):
    B, T = f0.shape
    d = W2.shape[1]

    bins, table = pl.pallas_call(
        _prep_body,
        out_shape=(
            jax.ShapeDtypeStruct((B, T), jnp.int32),
            jax.ShapeDtypeStruct((N_F0_BINS, d), jnp.float32),
        ),
    )(f0, emb, W1, b1.reshape(1, -1), W2, b2.reshape(1, -1))

    n_rows = B * T
    n_sc = 20480
    bins_flat = bins.reshape(n_rows)

    out_sc = _make_sc_gather(n_sc, d, chunk=32)(table, bins_flat[:n_sc])

    nt_all = n_rows // _TC_TILE
    nt_sc = n_sc // _TC_TILE
    nt = nt_all - nt_sc
    table_bf = table.astype(jnp.bfloat16)
    out_full = pl.pallas_call(
        _onehot_body,
        grid=(nt,),
        in_specs=[
            pl.BlockSpec((1, 1, _TC_TILE), lambda i: (i, 0, 0)),
            pl.BlockSpec((N_F0_BINS, d), lambda i: (0, 0)),
        ],
        out_specs=pl.BlockSpec((1, _TC_TILE, d), lambda i: (i + nt_sc, 0, 0)),
        out_shape=jax.ShapeDtypeStruct((nt_all, _TC_TILE, d), jnp.float32),
    )(bins_flat[n_sc:].reshape(nt, 1, _TC_TILE), table_bf)

    out = lax.dynamic_update_slice(out_full.reshape(n_rows, d), out_sc, (0, 0))
    return out.reshape(B, T, d)


# speedup vs baseline: 6.9485x; 1.1110x over previous
import functools
import math

import jax
import jax.numpy as jnp
from jax import lax
from jax.experimental import pallas as pl
from jax.experimental.pallas import tpu as pltpu
from jax.experimental.pallas import tpu_sc as plsc

N_F0_BINS = 256
V1_DIM = 768
HIDDEN_DIM = 512
F0_MIN = 50.0
F0_MAX = 1100.0

_MEL_MIN = 1127.0 * math.log(1.0 + F0_MIN / 700.0)
_MEL_MAX = 1127.0 * math.log(1.0 + F0_MAX / 700.0)

_NC = 2
_NS = 16
_NW = _NC * _NS


def _prep_body(f0_ref, emb_ref, w1_ref, b1_ref, w2_ref, b2_ref,
               bins_ref, table_ref):
    f0 = f0_ref[...]
    f0_mel = 1127.0 * jnp.log(1.0 + f0 / 700.0)
    f0_mel = jnp.where(
        f0_mel > 0.0,
        (f0_mel - _MEL_MIN) * (N_F0_BINS - 2) / (_MEL_MAX - _MEL_MIN) + 1.0,
        f0_mel,
    )
    f0_mel = jnp.where(f0_mel <= 1.0, 1.0, f0_mel)
    f0_mel = jnp.where(f0_mel > N_F0_BINS - 1, float(N_F0_BINS - 1), f0_mel)
    bins_ref[...] = (f0_mel + 0.5).astype(jnp.int32)

    h = jnp.dot(emb_ref[...], w1_ref[...], preferred_element_type=jnp.float32)
    h = h + b1_ref[...]
    h = 0.5 * h * (1.0 + lax.erf(h * (1.0 / math.sqrt(2.0))))
    t = jnp.dot(h, w2_ref[...], preferred_element_type=jnp.float32)
    table_ref[...] = t + b2_ref[...]


_NBUF = 4


def _make_sc_gather(n_rows, d, chunk):
    n_per_w = n_rows // _NW
    n_chunks = n_per_w // chunk
    assert n_chunks % _NBUF == 0
    mesh = plsc.VectorSubcoreMesh(core_axis_name="c", subcore_axis_name="s")

    @functools.partial(
        pl.kernel,
        mesh=mesh,
        out_type=jax.ShapeDtypeStruct((n_rows, d), jnp.float32),
        scratch_types=[
            pltpu.VMEM((n_per_w,), jnp.int32),
            pltpu.VMEM((_NBUF, chunk, d), jnp.float32),
        ]
        + [pltpu.SemaphoreType.DMA] * (2 * _NBUF),
    )
    def gather_kernel(table_hbm, bins_hbm, out_hbm, idx_v, rows_v, *sems):
        gsems, ssems = sems[:_NBUF], sems[_NBUF:]
        wid = lax.axis_index("s") * _NC + lax.axis_index("c")
        base = wid * n_per_w
        pltpu.sync_copy(bins_hbm.at[pl.ds(base, n_per_w)], idx_v)

        def gather_chunk(off, b):
            return pltpu.make_async_copy(
                table_hbm.at[idx_v.at[pl.ds(off, chunk)]],
                rows_v.at[b], gsems[b])

        def store_chunk(off, b):
            return pltpu.make_async_copy(
                rows_v.at[b], out_hbm.at[pl.ds(base + off, chunk)], ssems[b])

        for b in range(_NBUF):
            gather_chunk(b * chunk, b).start()

        def body(j, _):
            for b in range(_NBUF):
                i = j * _NBUF + b
                off = i * chunk
                gather_chunk(off, b).wait()
                store_chunk(off, b).start()
                store_chunk(off, b).wait()

                @pl.when(j + 1 < n_chunks // _NBUF)
                def _():
                    gather_chunk(off + _NBUF * chunk, b).start()

            return 0

        lax.fori_loop(0, n_chunks // _NBUF, body, 0)

    return gather_kernel


_TC_TILE = 2048


def _onehot_body(bins_ref, table_ref, out_ref):
    b = bins_ref[0]
    onehot = (b[:, :, None] == lax.broadcasted_iota(
        jnp.int32, (1, _TC_TILE, N_F0_BINS), 2)).astype(jnp.bfloat16)
    out_ref[0] = jnp.dot(onehot[0], table_ref[...],
                         preferred_element_type=jnp.float32)


def kernel(f0, emb, W1, b1, W2, b2):
    B, T = f0.shape
    d = W2.shape[1]

    bins, table = pl.pallas_call(
        _prep_body,
        out_shape=(
            jax.ShapeDtypeStruct((B, T), jnp.int32),
            jax.ShapeDtypeStruct((N_F0_BINS, d), jnp.float32),
        ),
    )(f0, emb, W1, b1.reshape(1, -1), W2, b2.reshape(1, -1))

    n_rows = B * T
    n_sc = 16384
    bins_flat = bins.reshape(n_rows)

    out_sc = _make_sc_gather(n_sc, d, chunk=32)(table, bins_flat[:n_sc])

    nt_all = n_rows // _TC_TILE
    nt_sc = n_sc // _TC_TILE
    nt = nt_all - nt_sc
    table_bf = table.astype(jnp.bfloat16)
    out_full = pl.pallas_call(
        _onehot_body,
        grid=(nt,),
        in_specs=[
            pl.BlockSpec((1, 1, _TC_TILE), lambda i: (i, 0, 0)),
            pl.BlockSpec((N_F0_BINS, d), lambda i: (0, 0)),
        ],
        out_specs=pl.BlockSpec((1, _TC_TILE, d), lambda i: (i + nt_sc, 0, 0)),
        out_shape=jax.ShapeDtypeStruct((nt_all, _TC_TILE, d), jnp.float32),
    )(bins_flat[n_sc:].reshape(nt, 1, _TC_TILE), table_bf)

    out = lax.dynamic_update_slice(out_full.reshape(n_rows, d), out_sc, (0, 0))
    return out.reshape(B, T, d)


# speedup vs baseline: 7.2021x; 1.0365x over previous
import functools
import math

import jax
import jax.numpy as jnp
from jax import lax
from jax.experimental import pallas as pl
from jax.experimental.pallas import tpu as pltpu
from jax.experimental.pallas import tpu_sc as plsc

N_F0_BINS = 256
V1_DIM = 768
HIDDEN_DIM = 512
F0_MIN = 50.0
F0_MAX = 1100.0

_MEL_MIN = 1127.0 * math.log(1.0 + F0_MIN / 700.0)
_MEL_MAX = 1127.0 * math.log(1.0 + F0_MAX / 700.0)

_NC = 2
_NS = 16
_NW = _NC * _NS


def _prep_body(emb_ref, w1_ref, b1_ref, w2_ref, b2_ref, table_ref):
    h = jnp.dot(emb_ref[...], w1_ref[...], preferred_element_type=jnp.float32)
    h = h + b1_ref[...]
    h = 0.5 * h * (1.0 + lax.erf(h * (1.0 / math.sqrt(2.0))))
    t = jnp.dot(h, w2_ref[...], preferred_element_type=jnp.float32)
    table_ref[...] = t + b2_ref[...]


def _quantize_bins(f0):
    f0_mel = 1127.0 * jnp.log(1.0 + f0 / 700.0)
    f0_mel = jnp.where(
        f0_mel > 0.0,
        (f0_mel - _MEL_MIN) * (N_F0_BINS - 2) / (_MEL_MAX - _MEL_MIN) + 1.0,
        f0_mel,
    )
    f0_mel = jnp.where(f0_mel <= 1.0, 1.0, f0_mel)
    f0_mel = jnp.where(f0_mel > N_F0_BINS - 1, float(N_F0_BINS - 1), f0_mel)
    return (f0_mel + 0.5).astype(jnp.int32)


_NBUF = 4


def _make_sc_gather(n_rows, d, chunk):
    n_per_w = n_rows // _NW
    n_chunks = n_per_w // chunk
    assert n_chunks % _NBUF == 0
    mesh = plsc.VectorSubcoreMesh(core_axis_name="c", subcore_axis_name="s")

    @functools.partial(
        pl.kernel,
        mesh=mesh,
        out_type=jax.ShapeDtypeStruct((n_rows, d), jnp.float32),
        scratch_types=[
            pltpu.VMEM((n_per_w,), jnp.int32),
            pltpu.VMEM((_NBUF, chunk, d), jnp.float32),
        ]
        + [pltpu.SemaphoreType.DMA] * (2 * _NBUF),
    )
    def gather_kernel(table_hbm, bins_hbm, out_hbm, idx_v, rows_v, *sems):
        gsems, ssems = sems[:_NBUF], sems[_NBUF:]
        wid = lax.axis_index("s") * _NC + lax.axis_index("c")
        base = wid * n_per_w
        pltpu.sync_copy(bins_hbm.at[pl.ds(base, n_per_w)], idx_v)

        def gather_chunk(off, b):
            return pltpu.make_async_copy(
                table_hbm.at[idx_v.at[pl.ds(off, chunk)]],
                rows_v.at[b], gsems[b])

        def store_chunk(off, b):
            return pltpu.make_async_copy(
                rows_v.at[b], out_hbm.at[pl.ds(base + off, chunk)], ssems[b])

        for b in range(_NBUF):
            gather_chunk(b * chunk, b).start()

        def body(j, _):
            for b in range(_NBUF):
                i = j * _NBUF + b
                off = i * chunk
                gather_chunk(off, b).wait()
                store_chunk(off, b).start()
                store_chunk(off, b).wait()

                @pl.when(j + 1 < n_chunks // _NBUF)
                def _():
                    gather_chunk(off + _NBUF * chunk, b).start()

            return 0

        lax.fori_loop(0, n_chunks // _NBUF, body, 0)

    return gather_kernel


_TC_TILE = 2048


def _onehot_body(bins_ref, table_ref, out_ref):
    b = bins_ref[0]
    onehot = (b[:, :, None] == lax.broadcasted_iota(
        jnp.int32, (1, _TC_TILE, N_F0_BINS), 2)).astype(jnp.bfloat16)
    out_ref[0] = jnp.dot(onehot[0], table_ref[...],
                         preferred_element_type=jnp.float32)


def kernel(f0, emb, W1, b1, W2, b2):
    B, T = f0.shape
    d = W2.shape[1]

    table = pl.pallas_call(
        _prep_body,
        out_shape=jax.ShapeDtypeStruct((N_F0_BINS, d), jnp.float32),
    )(emb, W1, b1.reshape(1, -1), W2, b2.reshape(1, -1))
    bins = _quantize_bins(f0)

    n_rows = B * T
    n_sc = 16384
    bins_flat = bins.reshape(n_rows)

    out_sc = _make_sc_gather(n_sc, d, chunk=32)(table, bins_flat[:n_sc])

    nt_all = n_rows // _TC_TILE
    nt_sc = n_sc // _TC_TILE
    nt = nt_all - nt_sc
    table_bf = table.astype(jnp.bfloat16)
    out_full = pl.pallas_call(
        _onehot_body,
        grid=(nt,),
        in_specs=[
            pl.BlockSpec((1, 1, _TC_TILE), lambda i: (i, 0, 0)),
            pl.BlockSpec((N_F0_BINS, d), lambda i: (0, 0)),
        ],
        out_specs=pl.BlockSpec((1, _TC_TILE, d), lambda i: (i + nt_sc, 0, 0)),
        out_shape=jax.ShapeDtypeStruct((nt_all, _TC_TILE, d), jnp.float32),
    )(bins_flat[n_sc:].reshape(nt, 1, _TC_TILE), table_bf)

    out = lax.dynamic_update_slice(out_full.reshape(n_rows, d), out_sc, (0, 0))
    return out.reshape(B, T, d)
